# Initial kernel scaffold; baseline (speedup 1.0000x reference)
import jax
import jax.numpy as jnp
from jax.experimental import pallas as pl


def kernel(x, edge_index, edge_attr, W, b):
    raise NotImplementedError("write your pallas kernel here")


# speedup vs baseline: 5.4020x; 5.4020x over previous
import jax
import jax.numpy as jnp
from jax import lax
from jax.experimental import pallas as pl
from jax.experimental.pallas import tpu as pltpu
from jax.experimental.pallas import tpu_sc as plsc

N = 10000
E = 160000
D_IN = 256
D_OUT = 256
H = D_OUT // 2
NC = 2
NS = 16
ET = E // NS
B = 80
NB = ET // B
NP = 10240
RPT = NP // NS
FIN_CH = 32


def _mm_body(x_ref, w_ref, o_ref):
    o_ref[0] = jnp.dot(x_ref[...], w_ref[...],
                       preferred_element_type=jnp.float32)


def _matmul_halves(x, W):
    bn = 320
    return pl.pallas_call(
        _mm_body,
        grid=(NP // bn, 2),
        in_specs=[
            pl.BlockSpec((bn, D_IN), lambda i, c: (i, 0)),
            pl.BlockSpec((D_IN, H), lambda i, c: (0, c)),
        ],
        out_specs=pl.BlockSpec((1, bn, H), lambda i, c: (c, i, 0)),
        out_shape=jax.ShapeDtypeStruct((2, NP, H), jnp.float32),
    )(x, W)


def _bcast16(i):
    return jnp.zeros((16,), jnp.int32) + i


def _sc_body(xw2, srcp, dstp, ewp, b2, out,
             src_b, dst_b, ew_b, idxg_v, norm_v, diss_b, disd_b, disb_v,
             rows_v, accb_v, xwb_v, bias_v, w640_v, deg_sh, acc_sh):
    cid = lax.axis_index("c")
    sid = lax.axis_index("s")
    coff = cid * NP

    pltpu.sync_copy(b2.at[cid], bias_v)

    def _fill_ones(i, _):
        w640_v[pl.ds(i * 16, 16)] = jnp.ones((16,), jnp.float32)
        return 0
    lax.fori_loop(0, 640 // 16, _fill_ones, 0)
    pltpu.sync_copy(w640_v, deg_sh.at[pl.ds(sid * 640, 640)])

    def _zero_row(i, _):
        for k in range(H // 16):
            accb_v[i, pl.ds(k * 16, 16)] = jnp.zeros((16,), jnp.float32)
        return 0
    lax.fori_loop(0, FIN_CH, _zero_row, 0)
    for c in range(RPT // FIN_CH):
        pltpu.sync_copy(accb_v,
                        acc_sh.at[pl.ds(sid * RPT + c * FIN_CH, FIN_CH)])
    plsc.subcore_barrier()

    def _deg_batch(j, _):
        pltpu.sync_copy(dstp.at[sid, j], dst_b)
        pltpu.sync_copy(ewp.at[sid, j], ew_b)
        pltpu.sync_copy(ew_b, deg_sh.at[dst_b], add=True)
        return 0
    lax.fori_loop(0, NB, _deg_batch, 0)
    plsc.subcore_barrier()

    pltpu.sync_copy(deg_sh.at[pl.ds(sid * 640, 640)], w640_v)

    def _newton(i, _):
        d = w640_v[pl.ds(i * 16, 16)]
        bits = lax.bitcast_convert_type(d, jnp.int32)
        y = lax.bitcast_convert_type(jnp.int32(0x5F3759DF) - (bits >> 1),
                                     jnp.float32)
        hd = d * jnp.float32(-0.5)
        for _ in range(3):
            y = y * (hd * y * y + jnp.float32(1.5))
        w640_v[pl.ds(i * 16, 16)] = y
        return 0
    lax.fori_loop(0, 640 // 16, _newton, 0)
    pltpu.sync_copy(w640_v, deg_sh.at[pl.ds(sid * 640, 640)])
    plsc.subcore_barrier()

    def _edge_batch(j, _):
        pltpu.sync_copy(srcp.at[sid, j], src_b)
        pltpu.sync_copy(dstp.at[sid, j], dst_b)
        pltpu.sync_copy(ewp.at[sid, j], ew_b)
        pltpu.sync_copy(deg_sh.at[src_b], diss_b)
        pltpu.sync_copy(deg_sh.at[dst_b], disd_b)
        for k in range(B // 16):
            sl = pl.ds(k * 16, 16)
            norm_v[sl] = diss_b[sl] * ew_b[sl] * disd_b[sl]
            idxg_v[sl] = src_b[sl] + coff
        pltpu.sync_copy(xw2.at[idxg_v], rows_v)

        def _scale_row(i, _):
            nb = plsc.load_gather(norm_v, [_bcast16(i)])
            for k in range(H // 16):
                sl = pl.ds(k * 16, 16)
                rows_v[i, sl] = rows_v[i, sl] * nb
            return 0
        lax.fori_loop(0, B, _scale_row, 0)
        pltpu.sync_copy(rows_v, acc_sh.at[dst_b], add=True)
        return 0
    lax.fori_loop(0, NB, _edge_batch, 0)
    plsc.subcore_barrier()

    for c in range(RPT // FIN_CH):
        base = sid * RPT + c * FIN_CH
        pltpu.sync_copy(acc_sh.at[pl.ds(base, FIN_CH)], accb_v)
        pltpu.sync_copy(xw2.at[pl.ds(coff + base, FIN_CH)], xwb_v)
        pltpu.sync_copy(deg_sh.at[pl.ds(base, FIN_CH)], disb_v)

        def _fin_row(i, _):
            dsq = plsc.load_gather(disb_v, [_bcast16(i)])
            dsq = dsq * dsq
            for k in range(H // 16):
                sl = pl.ds(k * 16, 16)
                accb_v[i, sl] = accb_v[i, sl] + xwb_v[i, sl] * dsq + \
                    bias_v[sl]
            return 0
        lax.fori_loop(0, FIN_CH, _fin_row, 0)
        pltpu.sync_copy(accb_v,
                        out.at[pl.ds(base, FIN_CH), pl.ds(cid * H, H)])


@jax.jit
def kernel(x, edge_index, edge_attr, W, b):
    xw2 = _matmul_halves(x, W).reshape(2 * NP, H)
    srcp = edge_index[0].reshape(NS, NB, B)
    dstp = edge_index[1].reshape(NS, NB, B)
    ewp = edge_attr.reshape(NS, NB, B)
    b2 = b.reshape(2, H)

    mesh = plsc.VectorSubcoreMesh(core_axis_name="c", subcore_axis_name="s",
                                  num_cores=NC, num_subcores=NS)
    sc_fn = pl.kernel(
        _sc_body,
        out_type=jax.ShapeDtypeStruct((NP, D_OUT), jnp.float32),
        mesh=mesh,
        compiler_params=pltpu.CompilerParams(needs_layout_passes=False),
        scratch_types=[
            pltpu.VMEM((B,), jnp.int32),
            pltpu.VMEM((B,), jnp.int32),
            pltpu.VMEM((B,), jnp.float32),
            pltpu.VMEM((B,), jnp.int32),
            pltpu.VMEM((B,), jnp.float32),
            pltpu.VMEM((B,), jnp.float32),
            pltpu.VMEM((B,), jnp.float32),
            pltpu.VMEM((FIN_CH,), jnp.float32),
            pltpu.VMEM((B, H), jnp.float32),
            pltpu.VMEM((FIN_CH, H), jnp.float32),
            pltpu.VMEM((FIN_CH, H), jnp.float32),
            pltpu.VMEM((H,), jnp.float32),
            pltpu.VMEM((640,), jnp.float32),
            pltpu.VMEM_SHARED((NP,), jnp.float32),
            pltpu.VMEM_SHARED((NP, H), jnp.float32),
        ],
    )
    return sc_fn(xw2, srcp, dstp, ewp, b2)[:N]


# speedup vs baseline: 12.5280x; 2.3191x over previous
import jax
import jax.numpy as jnp
from jax import lax
from jax.experimental import pallas as pl
from jax.experimental.pallas import tpu as pltpu
from jax.experimental.pallas import tpu_sc as plsc

N = 10000
E = 160000
D_IN = 256
D_OUT = 256
H = D_OUT // 2
NC = 2
NS = 16
ET = E // NS
B = 80
G = 25
NG = ET // (G * B)
NP = 10240
RPT = NP // NS


def _mm_body(x_ref, w_ref, o_ref):
    o_ref[0] = jnp.dot(x_ref[...], w_ref[...],
                       preferred_element_type=jnp.float32)


def _matmul_halves(x, W):
    bn = 320
    return pl.pallas_call(
        _mm_body,
        grid=(NP // bn, 2),
        in_specs=[
            pl.BlockSpec((bn, D_IN), lambda i, c: (i, 0)),
            pl.BlockSpec((D_IN, H), lambda i, c: (0, c)),
        ],
        out_specs=pl.BlockSpec((1, bn, H), lambda i, c: (c, i, 0)),
        out_shape=jax.ShapeDtypeStruct((2, NP, H), jnp.float32),
    )(x, W)


def _bcast16(i):
    return jnp.zeros((16,), jnp.int32) + i


def _sc_body(xw2, srcp, dstp, ewp, b2, out,
             srcg, dstg, nrmg, disd, rows0, rows1, disb_v, bias_v,
             deg_sh, acc_sh, gsem, ssem, psem):
    cid = lax.axis_index("c")
    sid = lax.axis_index("s")
    coff = cid * NP
    rows = (rows0, rows1)

    pltpu.sync_copy(b2.at[cid], bias_v)

    def _fill_ones(i, _):
        disb_v[pl.ds(i * 16, 16)] = jnp.ones((16,), jnp.float32)
        return 0
    lax.fori_loop(0, B // 16, _fill_ones, 0)
    for k in range(RPT // B):
        pltpu.sync_copy(disb_v, deg_sh.at[pl.ds(sid * RPT + k * B, B)])

    def _zero_row(i, _):
        for k in range(H // 16):
            rows0[i, pl.ds(k * 16, 16)] = jnp.zeros((16,), jnp.float32)
        return 0
    lax.fori_loop(0, B, _zero_row, 0)
    for k in range(RPT // B):
        pltpu.sync_copy(rows0, acc_sh.at[pl.ds(sid * RPT + k * B, B)])
    plsc.subcore_barrier()

    def _deg_group(g, _):
        pltpu.sync_copy(dstp.at[sid, g], dstg)
        pltpu.sync_copy(ewp.at[sid, g], nrmg)
        descs = [pltpu.async_copy(nrmg.at[j], deg_sh.at[dstg.at[j]], psem,
                                  add=True)
                 for j in range(G)]
        for d in descs:
            d.wait()
        return 0
    lax.fori_loop(0, NG, _deg_group, 0)
    plsc.subcore_barrier()

    for k in range(RPT // B):
        off = sid * RPT + k * B
        pltpu.sync_copy(deg_sh.at[pl.ds(off, B)], disb_v)

        def _newton(i, _):
            d = disb_v[pl.ds(i * 16, 16)]
            bits = lax.bitcast_convert_type(d, jnp.int32)
            y = lax.bitcast_convert_type(
                jnp.int32(0x5F3759DF) - (bits >> 1), jnp.float32)
            hd = d * jnp.float32(-0.5)
            for _ in range(3):
                y = y * (hd * y * y + jnp.float32(1.5))
            disb_v[pl.ds(i * 16, 16)] = y
            return 0
        lax.fori_loop(0, B // 16, _newton, 0)
        pltpu.sync_copy(disb_v, deg_sh.at[pl.ds(off, B)])
    plsc.subcore_barrier()

    def _group(g, _):
        pltpu.sync_copy(srcp.at[sid, g], srcg)
        pltpu.sync_copy(dstp.at[sid, g], dstg)
        pltpu.sync_copy(ewp.at[sid, g], nrmg)

        descs = [pltpu.async_copy(deg_sh.at[dstg.at[j]], disd.at[j], psem)
                 for j in range(G)]
        for d in descs:
            d.wait()

        def _mul_dst(j, _):
            for k in range(B // 16):
                sl = pl.ds(k * 16, 16)
                nrmg[j, sl] = nrmg[j, sl] * disd[j, sl]
            return 0
        lax.fori_loop(0, G, _mul_dst, 0)

        descs = [pltpu.async_copy(deg_sh.at[srcg.at[j]], disd.at[j], psem)
                 for j in range(G)]
        for d in descs:
            d.wait()

        def _mul_src(j, _):
            for k in range(B // 16):
                sl = pl.ds(k * 16, 16)
                nrmg[j, sl] = nrmg[j, sl] * disd[j, sl]
                srcg[j, sl] = srcg[j, sl] + coff
            return 0
        lax.fori_loop(0, G, _mul_src, 0)

        def _scale(j, buf):
            def _scale_row(i, _):
                nb = plsc.load_gather(nrmg, [_bcast16(j), _bcast16(i)])
                for k in range(H // 16):
                    sl = pl.ds(k * 16, 16)
                    buf[i, sl] = buf[i, sl] * nb
                return 0
            lax.fori_loop(0, B, _scale_row, 0)

        gat = {0: pltpu.async_copy(xw2.at[srcg.at[0]], rows[0], gsem)}
        sca = {}
        for j in range(G):
            cur = rows[j % 2]
            gat[j].wait()
            if j + 1 < G:
                if j - 1 >= 0:
                    sca[j - 1].wait()
                gat[j + 1] = pltpu.async_copy(
                    xw2.at[srcg.at[j + 1]], rows[(j + 1) % 2], gsem)
            _scale(j, cur)
            sca[j] = pltpu.async_copy(cur, acc_sh.at[dstg.at[j]], ssem,
                                      add=True)
        sca[G - 2].wait()
        sca[G - 1].wait()
        return 0
    lax.fori_loop(0, NG, _group, 0)
    plsc.subcore_barrier()

    for c in range(RPT // B):
        base = sid * RPT + c * B
        pltpu.sync_copy(acc_sh.at[pl.ds(base, B)], rows0)
        pltpu.sync_copy(xw2.at[pl.ds(coff + base, B)], rows1)
        pltpu.sync_copy(deg_sh.at[pl.ds(base, B)], disb_v)

        def _fin_row(i, _):
            dsq = plsc.load_gather(disb_v, [_bcast16(i)])
            dsq = dsq * dsq
            for k in range(H // 16):
                sl = pl.ds(k * 16, 16)
                rows0[i, sl] = rows0[i, sl] + rows1[i, sl] * dsq + \
                    bias_v[sl]
            return 0
        lax.fori_loop(0, B, _fin_row, 0)
        pltpu.sync_copy(rows0, out.at[pl.ds(base, B), pl.ds(cid * H, H)])


@jax.jit
def kernel(x, edge_index, edge_attr, W, b):
    xw2 = _matmul_halves(x, W).reshape(2 * NP, H)
    srcp = edge_index[0].reshape(NS, NG, G, B)
    dstp = edge_index[1].reshape(NS, NG, G, B)
    ewp = edge_attr.reshape(NS, NG, G, B)
    b2 = b.reshape(2, H)

    mesh = plsc.VectorSubcoreMesh(core_axis_name="c", subcore_axis_name="s",
                                  num_cores=NC, num_subcores=NS)
    sc_fn = pl.kernel(
        _sc_body,
        out_type=jax.ShapeDtypeStruct((NP, D_OUT), jnp.float32),
        mesh=mesh,
        compiler_params=pltpu.CompilerParams(needs_layout_passes=False),
        scratch_types=[
            pltpu.VMEM((G, B), jnp.int32),
            pltpu.VMEM((G, B), jnp.int32),
            pltpu.VMEM((G, B), jnp.float32),
            pltpu.VMEM((G, B), jnp.float32),
            pltpu.VMEM((B, H), jnp.float32),
            pltpu.VMEM((B, H), jnp.float32),
            pltpu.VMEM((B,), jnp.float32),
            pltpu.VMEM((H,), jnp.float32),
            pltpu.VMEM_SHARED((NP,), jnp.float32),
            pltpu.VMEM_SHARED((NP, H), jnp.float32),
            pltpu.SemaphoreType.DMA,
            pltpu.SemaphoreType.DMA,
            pltpu.SemaphoreType.DMA,
        ],
    )
    return sc_fn(xw2, srcp, dstp, ewp, b2)[:N]


# speedup vs baseline: 14.6375x; 1.1684x over previous
import jax
import jax.numpy as jnp
from jax import lax
from jax.experimental import pallas as pl
from jax.experimental.pallas import tpu as pltpu
from jax.experimental.pallas import tpu_sc as plsc

N = 10000
E = 160000
D_IN = 256
D_OUT = 256
H = D_OUT // 2
NC = 2
NS = 16
ET = E // NS
B = 80
G = 25
NG = ET // (G * B)
NP = 10240
RPT = NP // NS
SB = RPT // B


def _mm_body(x_ref, w_ref, o_ref):
    o_ref[0] = jnp.dot(x_ref[...], w_ref[...],
                       preferred_element_type=jnp.float32)


def _matmul_halves(x, W):
    bn = 320
    return pl.pallas_call(
        _mm_body,
        grid=(NP // bn, 2),
        in_specs=[
            pl.BlockSpec((bn, D_IN), lambda i, c: (i, 0)),
            pl.BlockSpec((D_IN, H), lambda i, c: (0, c)),
        ],
        out_specs=pl.BlockSpec((1, bn, H), lambda i, c: (c, i, 0)),
        out_shape=jax.ShapeDtypeStruct((2, NP, H), jnp.float32),
    )(x, W)


def _bcast16(i):
    return jnp.zeros((16,), jnp.int32) + i


def _sc_body(xw2, srcp, dstp, ewp, b2, out,
             srcg, dstg, nrmg, disd, rows0, rows1, disb_v, bias_v,
             deg_sh, acc_sh, gsem, ssem, psem):
    cid = lax.axis_index("c")
    sid = lax.axis_index("s")
    coff = cid * NP
    rows = (rows0, rows1)

    pltpu.sync_copy(b2.at[cid], bias_v)

    def _fill_ones(i, _):
        disb_v[pl.ds(i * 16, 16)] = jnp.ones((16,), jnp.float32)
        return 0
    lax.fori_loop(0, B // 16, _fill_ones, 0)
    for k in range(SB):
        pltpu.sync_copy(disb_v, deg_sh.at[pl.ds(sid * RPT + k * B, B)])

    def _bias_row(i, _):
        for k in range(H // 16):
            sl = pl.ds(k * 16, 16)
            rows0[i, sl] = bias_v[sl]
        return 0
    lax.fori_loop(0, B, _bias_row, 0)
    for k in range(SB):
        pltpu.sync_copy(rows0, acc_sh.at[pl.ds(sid * RPT + k * B, B)])
    plsc.subcore_barrier()

    def _deg_group(g, _):
        d1 = pltpu.async_copy(dstp.at[sid, g], dstg, psem)
        d2 = pltpu.async_copy(ewp.at[sid, g], nrmg, psem)
        d1.wait()
        d2.wait()
        descs = [pltpu.async_copy(nrmg.at[j], deg_sh.at[dstg.at[j]], psem,
                                  add=True)
                 for j in range(G)]
        for d in descs:
            d.wait()
        return 0
    lax.fori_loop(0, NG, _deg_group, 0)
    plsc.subcore_barrier()

    for k in range(SB):
        off = sid * RPT + k * B
        pltpu.sync_copy(deg_sh.at[pl.ds(off, B)], disb_v)

        def _newton(i, _):
            d = disb_v[pl.ds(i * 16, 16)]
            bits = lax.bitcast_convert_type(d, jnp.int32)
            y = lax.bitcast_convert_type(
                jnp.int32(0x5F3759DF) - (bits >> 1), jnp.float32)
            hd = d * jnp.float32(-0.5)
            for _ in range(3):
                y = y * (hd * y * y + jnp.float32(1.5))
            disb_v[pl.ds(i * 16, 16)] = y
            return 0
        lax.fori_loop(0, B // 16, _newton, 0)
        pltpu.sync_copy(disb_v, deg_sh.at[pl.ds(off, B)])
    plsc.subcore_barrier()

    def _run_pipeline(nb):
        def _scale(j, buf):
            def _scale_row(i, _):
                for u in range(2):
                    r = 2 * i + u
                    nb_ = plsc.load_gather(nrmg,
                                           [_bcast16(j), _bcast16(r)])
                    for k in range(H // 16):
                        sl = pl.ds(k * 16, 16)
                        buf[r, sl] = buf[r, sl] * nb_
                return 0
            lax.fori_loop(0, B // 2, _scale_row, 0)

        gat = {0: pltpu.async_copy(xw2.at[srcg.at[0]], rows[0], gsem)}
        sca = {}
        for j in range(nb):
            cur = rows[j % 2]
            gat[j].wait()
            if j + 1 < nb:
                if j - 1 >= 0:
                    sca[j - 1].wait()
                gat[j + 1] = pltpu.async_copy(
                    xw2.at[srcg.at[j + 1]], rows[(j + 1) % 2], gsem)
            _scale(j, cur)
            sca[j] = pltpu.async_copy(cur, acc_sh.at[dstg.at[j]], ssem,
                                      add=True)
        if nb >= 2:
            sca[nb - 2].wait()
        sca[nb - 1].wait()

    def _group(g, _):
        d1 = pltpu.async_copy(srcp.at[sid, g], srcg, psem)
        d2 = pltpu.async_copy(dstp.at[sid, g], dstg, psem)
        d3 = pltpu.async_copy(ewp.at[sid, g], nrmg, psem)
        d1.wait()
        d2.wait()
        d3.wait()

        descs = [pltpu.async_copy(deg_sh.at[dstg.at[j]], disd.at[j], psem)
                 for j in range(G)]
        for d in descs:
            d.wait()

        def _mul_dst(j, _):
            for k in range(B // 16):
                sl = pl.ds(k * 16, 16)
                nrmg[j, sl] = nrmg[j, sl] * disd[j, sl]
            return 0
        lax.fori_loop(0, G, _mul_dst, 0)

        descs = [pltpu.async_copy(deg_sh.at[srcg.at[j]], disd.at[j], psem)
                 for j in range(G)]
        for d in descs:
            d.wait()

        def _mul_src(j, _):
            for k in range(B // 16):
                sl = pl.ds(k * 16, 16)
                nrmg[j, sl] = nrmg[j, sl] * disd[j, sl]
                srcg[j, sl] = srcg[j, sl] + coff
            return 0
        lax.fori_loop(0, G, _mul_src, 0)

        _run_pipeline(G)
        return 0
    lax.fori_loop(0, NG, _group, 0)

    descs = [pltpu.async_copy(deg_sh.at[pl.ds(sid * RPT + j * B, B)],
                              disd.at[j], psem)
             for j in range(SB)]
    for d in descs:
        d.wait()

    def _self_stage(j, _):
        base = sid * RPT + j * B
        for k in range(B // 16):
            sl = pl.ds(k * 16, 16)
            idx = lax.iota(jnp.int32, 16) + (base + k * 16)
            dstg[j, sl] = idx
            srcg[j, sl] = idx + coff
            nrmg[j, sl] = disd[j, sl] * disd[j, sl]
        return 0
    lax.fori_loop(0, SB, _self_stage, 0)
    _run_pipeline(SB)
    plsc.subcore_barrier()

    pltpu.sync_copy(acc_sh.at[pl.ds(sid * RPT, RPT)],
                    out.at[pl.ds(sid * RPT, RPT), pl.ds(cid * H, H)])


@jax.jit
def kernel(x, edge_index, edge_attr, W, b):
    xw2 = _matmul_halves(x, W).reshape(2 * NP, H)
    srcp = edge_index[0].reshape(NS, NG, G, B)
    dstp = edge_index[1].reshape(NS, NG, G, B)
    ewp = edge_attr.reshape(NS, NG, G, B)
    b2 = b.reshape(2, H)

    mesh = plsc.VectorSubcoreMesh(core_axis_name="c", subcore_axis_name="s",
                                  num_cores=NC, num_subcores=NS)
    sc_fn = pl.kernel(
        _sc_body,
        out_type=jax.ShapeDtypeStruct((NP, D_OUT), jnp.float32),
        mesh=mesh,
        compiler_params=pltpu.CompilerParams(needs_layout_passes=False),
        scratch_types=[
            pltpu.VMEM((G, B), jnp.int32),
            pltpu.VMEM((G, B), jnp.int32),
            pltpu.VMEM((G, B), jnp.float32),
            pltpu.VMEM((G, B), jnp.float32),
            pltpu.VMEM((B, H), jnp.float32),
            pltpu.VMEM((B, H), jnp.float32),
            pltpu.VMEM((B,), jnp.float32),
            pltpu.VMEM((H,), jnp.float32),
            pltpu.VMEM_SHARED((NP,), jnp.float32),
            pltpu.VMEM_SHARED((NP, H), jnp.float32),
            pltpu.SemaphoreType.DMA,
            pltpu.SemaphoreType.DMA,
            pltpu.SemaphoreType.DMA,
        ],
    )
    return sc_fn(xw2, srcp, dstp, ewp, b2)[:N]


# speedup vs baseline: 14.6550x; 1.0012x over previous
import jax
import jax.numpy as jnp
from jax import lax
from jax.experimental import pallas as pl
from jax.experimental.pallas import tpu as pltpu
from jax.experimental.pallas import tpu_sc as plsc

N = 10000
E = 160000
D_IN = 256
D_OUT = 256
H = D_OUT // 2
NC = 2
NS = 16
ET = E // NS
B = 80
G = 25
NG = ET // (G * B)
NP = 10240
RPT = NP // NS
SB = RPT // B


def _mm_body(x_ref, w_ref, o_ref):
    o_ref[0] = jnp.dot(x_ref[...], w_ref[...],
                       preferred_element_type=jnp.float32)


def _matmul_halves(x, W):
    bn = 320
    return pl.pallas_call(
        _mm_body,
        grid=(NP // bn, 2),
        in_specs=[
            pl.BlockSpec((bn, D_IN), lambda i, c: (i, 0)),
            pl.BlockSpec((D_IN, H), lambda i, c: (0, c)),
        ],
        out_specs=pl.BlockSpec((1, bn, H), lambda i, c: (c, i, 0)),
        out_shape=jax.ShapeDtypeStruct((2, NP, H), jnp.float32),
    )(x, W)


def _bcast16(i):
    return jnp.zeros((16,), jnp.int32) + i


def _sc_body(xw2, srcp, dstp, ewp, b2, out,
             srcg, dstg, nrmg, disd, rows0, rows1, disb_v, bias_v,
             deg_sh, acc_sh, gsem, ssem, psem):
    cid = lax.axis_index("c")
    sid = lax.axis_index("s")
    coff = cid * NP
    rows = (rows0, rows1)

    pltpu.sync_copy(b2.at[cid], bias_v)

    def _fill_ones(i, _):
        disb_v[pl.ds(i * 16, 16)] = jnp.ones((16,), jnp.float32)
        return 0
    lax.fori_loop(0, B // 16, _fill_ones, 0)
    for k in range(SB):
        pltpu.sync_copy(disb_v, deg_sh.at[pl.ds(sid * RPT + k * B, B)])

    def _bias_row(i, _):
        for k in range(H // 16):
            sl = pl.ds(k * 16, 16)
            rows0[i, sl] = bias_v[sl]
        return 0
    lax.fori_loop(0, B, _bias_row, 0)
    for k in range(SB):
        pltpu.sync_copy(rows0, acc_sh.at[pl.ds(sid * RPT + k * B, B)])
    plsc.subcore_barrier()

    scope_deg = jax.named_scope("deg_phase")
    scope_deg.__enter__()
    def _deg_group(g, _):
        d1 = pltpu.async_copy(dstp.at[sid, g], dstg, psem)
        d2 = pltpu.async_copy(ewp.at[sid, g], nrmg, psem)
        d1.wait()
        d2.wait()
        descs = [pltpu.async_copy(nrmg.at[j], deg_sh.at[dstg.at[j]], psem,
                                  add=True)
                 for j in range(G)]
        for d in descs:
            d.wait()
        return 0
    lax.fori_loop(0, NG, _deg_group, 0)
    plsc.subcore_barrier()
    scope_deg.__exit__(None, None, None)

    scope_newton = jax.named_scope("newton_phase")
    scope_newton.__enter__()
    for k in range(SB):
        off = sid * RPT + k * B
        pltpu.sync_copy(deg_sh.at[pl.ds(off, B)], disb_v)

        def _newton(i, _):
            d = disb_v[pl.ds(i * 16, 16)]
            bits = lax.bitcast_convert_type(d, jnp.int32)
            y = lax.bitcast_convert_type(
                jnp.int32(0x5F3759DF) - (bits >> 1), jnp.float32)
            hd = d * jnp.float32(-0.5)
            for _ in range(3):
                y = y * (hd * y * y + jnp.float32(1.5))
            disb_v[pl.ds(i * 16, 16)] = y
            return 0
        lax.fori_loop(0, B // 16, _newton, 0)
        pltpu.sync_copy(disb_v, deg_sh.at[pl.ds(off, B)])
    plsc.subcore_barrier()
    scope_newton.__exit__(None, None, None)

    def _run_pipeline(nb):
        def _scale(j, buf):
            def _scale_row(i, _):
                for u in range(2):
                    r = 2 * i + u
                    nb_ = plsc.load_gather(nrmg,
                                           [_bcast16(j), _bcast16(r)])
                    for k in range(H // 16):
                        sl = pl.ds(k * 16, 16)
                        buf[r, sl] = buf[r, sl] * nb_
                return 0
            lax.fori_loop(0, B // 2, _scale_row, 0)

        gat = {0: pltpu.async_copy(xw2.at[srcg.at[0]], rows[0], gsem)}
        sca = {}
        for j in range(nb):
            cur = rows[j % 2]
            gat[j].wait()
            if j + 1 < nb:
                if j - 1 >= 0:
                    sca[j - 1].wait()
                gat[j + 1] = pltpu.async_copy(
                    xw2.at[srcg.at[j + 1]], rows[(j + 1) % 2], gsem)
            _scale(j, cur)
            sca[j] = pltpu.async_copy(cur, acc_sh.at[dstg.at[j]], ssem,
                                      add=True)
        if nb >= 2:
            sca[nb - 2].wait()
        sca[nb - 1].wait()

    def _group(g, _):
        d1 = pltpu.async_copy(srcp.at[sid, g], srcg, psem)
        d2 = pltpu.async_copy(dstp.at[sid, g], dstg, psem)
        d3 = pltpu.async_copy(ewp.at[sid, g], nrmg, psem)
        d1.wait()
        d2.wait()
        d3.wait()

        descs = [pltpu.async_copy(deg_sh.at[dstg.at[j]], disd.at[j], psem)
                 for j in range(G)]
        for d in descs:
            d.wait()

        def _mul_dst(j, _):
            for k in range(B // 16):
                sl = pl.ds(k * 16, 16)
                nrmg[j, sl] = nrmg[j, sl] * disd[j, sl]
            return 0
        lax.fori_loop(0, G, _mul_dst, 0)

        descs = [pltpu.async_copy(deg_sh.at[srcg.at[j]], disd.at[j], psem)
                 for j in range(G)]
        for d in descs:
            d.wait()

        def _mul_src(j, _):
            for k in range(B // 16):
                sl = pl.ds(k * 16, 16)
                nrmg[j, sl] = nrmg[j, sl] * disd[j, sl]
                srcg[j, sl] = srcg[j, sl] + coff
            return 0
        lax.fori_loop(0, G, _mul_src, 0)

        _run_pipeline(G)
        return 0
    scope_main = jax.named_scope("main_phase")
    scope_main.__enter__()
    lax.fori_loop(0, NG, _group, 0)
    scope_main.__exit__(None, None, None)

    scope_self = jax.named_scope("self_phase")
    scope_self.__enter__()
    descs = [pltpu.async_copy(deg_sh.at[pl.ds(sid * RPT + j * B, B)],
                              disd.at[j], psem)
             for j in range(SB)]
    for d in descs:
        d.wait()

    def _self_stage(j, _):
        base = sid * RPT + j * B
        for k in range(B // 16):
            sl = pl.ds(k * 16, 16)
            idx = lax.iota(jnp.int32, 16) + (base + k * 16)
            dstg[j, sl] = idx
            srcg[j, sl] = idx + coff
            nrmg[j, sl] = disd[j, sl] * disd[j, sl]
        return 0
    lax.fori_loop(0, SB, _self_stage, 0)
    _run_pipeline(SB)
    plsc.subcore_barrier()
    scope_self.__exit__(None, None, None)

    pltpu.sync_copy(acc_sh.at[pl.ds(sid * RPT, RPT)],
                    out.at[pl.ds(sid * RPT, RPT), pl.ds(cid * H, H)])


@jax.jit
def kernel(x, edge_index, edge_attr, W, b):
    xw2 = _matmul_halves(x, W).reshape(2 * NP, H)
    srcp = edge_index[0].reshape(NS, NG, G, B)
    dstp = edge_index[1].reshape(NS, NG, G, B)
    ewp = edge_attr.reshape(NS, NG, G, B)
    b2 = b.reshape(2, H)

    mesh = plsc.VectorSubcoreMesh(core_axis_name="c", subcore_axis_name="s",
                                  num_cores=NC, num_subcores=NS)
    sc_fn = pl.kernel(
        _sc_body,
        out_type=jax.ShapeDtypeStruct((NP, D_OUT), jnp.float32),
        mesh=mesh,
        compiler_params=pltpu.CompilerParams(needs_layout_passes=False),
        scratch_types=[
            pltpu.VMEM((G, B), jnp.int32),
            pltpu.VMEM((G, B), jnp.int32),
            pltpu.VMEM((G, B), jnp.float32),
            pltpu.VMEM((G, B), jnp.float32),
            pltpu.VMEM((B, H), jnp.float32),
            pltpu.VMEM((B, H), jnp.float32),
            pltpu.VMEM((B,), jnp.float32),
            pltpu.VMEM((H,), jnp.float32),
            pltpu.VMEM_SHARED((NP,), jnp.float32),
            pltpu.VMEM_SHARED((NP, H), jnp.float32),
            pltpu.SemaphoreType.DMA,
            pltpu.SemaphoreType.DMA,
            pltpu.SemaphoreType.DMA,
        ],
    )
    return sc_fn(xw2, srcp, dstp, ewp, b2)[:N]


# speedup vs baseline: 15.2763x; 1.0424x over previous
import jax
import jax.numpy as jnp
from jax import lax
from jax.experimental import pallas as pl
from jax.experimental.pallas import tpu as pltpu
from jax.experimental.pallas import tpu_sc as plsc

N = 10000
E = 160000
D_IN = 256
D_OUT = 256
H = D_OUT // 2
NC = 2
NS = 16
ET = E // NS
B = 80
G = 25
NG = ET // (G * B)
NP = 10240
RPT = NP // NS
SB = RPT // B
LAST = N - 15 * RPT


def _mm_body(x_ref, w_ref, o_ref):
    o_ref[0] = jnp.dot(x_ref[...], w_ref[...],
                       preferred_element_type=jnp.float32)


def _matmul_halves(x, W):
    bn = 320
    return pl.pallas_call(
        _mm_body,
        grid=(NP // bn, 2),
        in_specs=[
            pl.BlockSpec((bn, D_IN), lambda i, c: (i, 0)),
            pl.BlockSpec((D_IN, H), lambda i, c: (0, c)),
        ],
        out_specs=pl.BlockSpec((1, bn, H), lambda i, c: (c, i, 0)),
        out_shape=jax.ShapeDtypeStruct((2, NP, H), jnp.float32),
    )(x, W)


def _bcast16(i):
    return jnp.zeros((16,), jnp.int32) + i


def _sc_body(xw2, srcp, dstp, ewp, b2, out,
             srcg, dstg, nrmg, disd, rows0, rows1, rows2, disb_v, bias_v,
             deg_sh, acc_sh, gsem, ssem, psem):
    cid = lax.axis_index("c")
    sid = lax.axis_index("s")
    coff = cid * NP
    rows = (rows0, rows1, rows2)

    pltpu.sync_copy(b2.at[cid], bias_v)

    def _fill_ones(i, _):
        disb_v[pl.ds(i * 16, 16)] = jnp.ones((16,), jnp.float32)
        return 0
    lax.fori_loop(0, B // 16, _fill_ones, 0)
    for k in range(SB):
        pltpu.sync_copy(disb_v, deg_sh.at[pl.ds(sid * RPT + k * B, B)])

    def _bias_row(i, _):
        for k in range(H // 16):
            sl = pl.ds(k * 16, 16)
            rows0[i, sl] = bias_v[sl]
        return 0
    lax.fori_loop(0, B, _bias_row, 0)
    for k in range(SB):
        pltpu.sync_copy(rows0, acc_sh.at[pl.ds(sid * RPT + k * B, B)])
    plsc.subcore_barrier()

    def _deg_group(g, _):
        d1 = pltpu.async_copy(dstp.at[sid, g], dstg, psem)
        d2 = pltpu.async_copy(ewp.at[sid, g], nrmg, psem)
        d1.wait()
        d2.wait()
        descs = [pltpu.async_copy(nrmg.at[j], deg_sh.at[dstg.at[j]], psem,
                                  add=True)
                 for j in range(G)]
        for d in descs:
            d.wait()
        return 0
    lax.fori_loop(0, NG, _deg_group, 0)
    plsc.subcore_barrier()

    for k in range(SB):
        off = sid * RPT + k * B
        pltpu.sync_copy(deg_sh.at[pl.ds(off, B)], disb_v)

        def _newton(i, _):
            d = disb_v[pl.ds(i * 16, 16)]
            bits = lax.bitcast_convert_type(d, jnp.int32)
            y = lax.bitcast_convert_type(
                jnp.int32(0x5F3759DF) - (bits >> 1), jnp.float32)
            hd = d * jnp.float32(-0.5)
            for _ in range(3):
                y = y * (hd * y * y + jnp.float32(1.5))
            disb_v[pl.ds(i * 16, 16)] = y
            return 0
        lax.fori_loop(0, B // 16, _newton, 0)
        pltpu.sync_copy(disb_v, deg_sh.at[pl.ds(off, B)])
    plsc.subcore_barrier()

    def _run_pipeline(nb):
        def _scale(j, buf):
            def _scale_row(i, _):
                for u in range(4):
                    r = 4 * i + u
                    nb_ = plsc.load_gather(nrmg,
                                           [_bcast16(j), _bcast16(r)])
                    for k in range(H // 16):
                        sl = pl.ds(k * 16, 16)
                        buf[r, sl] = buf[r, sl] * nb_
                return 0
            lax.fori_loop(0, B // 4, _scale_row, 0)

        gat = {0: pltpu.async_copy(xw2.at[srcg.at[0]], rows[0], gsem)}
        if nb > 1:
            gat[1] = pltpu.async_copy(xw2.at[srcg.at[1]], rows[1], gsem)
        sca = {}
        for j in range(nb):
            cur = rows[j % 3]
            gat[j].wait()
            if j + 2 < nb:
                if j - 1 >= 0:
                    sca[j - 1].wait()
                gat[j + 2] = pltpu.async_copy(
                    xw2.at[srcg.at[j + 2]], rows[(j + 2) % 3], gsem)
            _scale(j, cur)
            sca[j] = pltpu.async_copy(cur, acc_sh.at[dstg.at[j]], ssem,
                                      add=True)
        for j in range(max(0, nb - 3), nb):
            sca[j].wait()

    def _group(g, _):
        d1 = pltpu.async_copy(srcp.at[sid, g], srcg, psem)
        d2 = pltpu.async_copy(dstp.at[sid, g], dstg, psem)
        d3 = pltpu.async_copy(ewp.at[sid, g], nrmg, psem)
        d1.wait()
        d2.wait()
        d3.wait()

        descs = [pltpu.async_copy(deg_sh.at[dstg.at[j]], disd.at[j], psem)
                 for j in range(G)]
        for d in descs:
            d.wait()

        def _mul_dst(j, _):
            for k in range(B // 16):
                sl = pl.ds(k * 16, 16)
                nrmg[j, sl] = nrmg[j, sl] * disd[j, sl]
            return 0
        lax.fori_loop(0, G, _mul_dst, 0)

        descs = [pltpu.async_copy(deg_sh.at[srcg.at[j]], disd.at[j], psem)
                 for j in range(G)]
        for d in descs:
            d.wait()

        def _mul_src(j, _):
            for k in range(B // 16):
                sl = pl.ds(k * 16, 16)
                nrmg[j, sl] = nrmg[j, sl] * disd[j, sl]
                srcg[j, sl] = srcg[j, sl] + coff
            return 0
        lax.fori_loop(0, G, _mul_src, 0)

        _run_pipeline(G)
        return 0
    lax.fori_loop(0, NG, _group, 0)

    descs = [pltpu.async_copy(deg_sh.at[pl.ds(sid * RPT + j * B, B)],
                              disd.at[j], psem)
             for j in range(SB)]
    for d in descs:
        d.wait()

    def _self_stage(j, _):
        base = sid * RPT + j * B
        for k in range(B // 16):
            sl = pl.ds(k * 16, 16)
            idx = lax.iota(jnp.int32, 16) + (base + k * 16)
            dstg[j, sl] = idx
            srcg[j, sl] = idx + coff
            nrmg[j, sl] = disd[j, sl] * disd[j, sl]
        return 0
    lax.fori_loop(0, SB, _self_stage, 0)
    _run_pipeline(SB)
    plsc.subcore_barrier()

    @pl.when(sid < NS - 1)
    def _full_tile():
        pltpu.sync_copy(acc_sh.at[pl.ds(sid * RPT, RPT)],
                        out.at[pl.ds(sid * RPT, RPT), pl.ds(cid * H, H)])

    @pl.when(sid == NS - 1)
    def _last_tile():
        pltpu.sync_copy(acc_sh.at[pl.ds((NS - 1) * RPT, LAST)],
                        out.at[pl.ds((NS - 1) * RPT, LAST),
                               pl.ds(cid * H, H)])


@jax.jit
def kernel(x, edge_index, edge_attr, W, b):
    xw2 = _matmul_halves(x, W).reshape(2 * NP, H)
    srcp = edge_index[0].reshape(NS, NG, G, B)
    dstp = edge_index[1].reshape(NS, NG, G, B)
    ewp = edge_attr.reshape(NS, NG, G, B)
    b2 = b.reshape(2, H)

    mesh = plsc.VectorSubcoreMesh(core_axis_name="c", subcore_axis_name="s",
                                  num_cores=NC, num_subcores=NS)
    sc_fn = pl.kernel(
        _sc_body,
        out_type=jax.ShapeDtypeStruct((N, D_OUT), jnp.float32),
        mesh=mesh,
        compiler_params=pltpu.CompilerParams(needs_layout_passes=False),
        scratch_types=[
            pltpu.VMEM((G, B), jnp.int32),
            pltpu.VMEM((G, B), jnp.int32),
            pltpu.VMEM((G, B), jnp.float32),
            pltpu.VMEM((G, B), jnp.float32),
            pltpu.VMEM((B, H), jnp.float32),
            pltpu.VMEM((B, H), jnp.float32),
            pltpu.VMEM((B, H), jnp.float32),
            pltpu.VMEM((B,), jnp.float32),
            pltpu.VMEM((H,), jnp.float32),
            pltpu.VMEM_SHARED((NP,), jnp.float32),
            pltpu.VMEM_SHARED((NP, H), jnp.float32),
            pltpu.SemaphoreType.DMA,
            pltpu.SemaphoreType.DMA,
            pltpu.SemaphoreType.DMA,
        ],
    )
    return sc_fn(xw2, srcp, dstp, ewp, b2)


# speedup vs baseline: 16.6717x; 1.0913x over previous
import jax
import jax.numpy as jnp
from jax import lax
from jax.experimental import pallas as pl
from jax.experimental.pallas import tpu as pltpu
from jax.experimental.pallas import tpu_sc as plsc

N = 10000
E = 160000
D_IN = 256
D_OUT = 256
H = D_OUT // 2
NC = 2
NS = 16
ET = E // NS
B = 80
G = 25
NG = ET // (G * B)
NP = 10240
RPT = NP // NS
SB = RPT // B
LAST = N - 15 * RPT


def _mm_body(x_ref, w_ref, o_ref):
    o_ref[0] = jnp.dot(x_ref[...], w_ref[...],
                       preferred_element_type=jnp.float32)


def _matmul_halves(x, W):
    bn = 1024
    return pl.pallas_call(
        _mm_body,
        grid=(NP // bn, 2),
        in_specs=[
            pl.BlockSpec((bn, D_IN), lambda i, c: (i, 0)),
            pl.BlockSpec((D_IN, H), lambda i, c: (0, c)),
        ],
        out_specs=pl.BlockSpec((1, bn, H), lambda i, c: (c, i, 0)),
        out_shape=jax.ShapeDtypeStruct((2, NP, H), jnp.float32),
    )(x, W)


def _bcast16(i):
    return jnp.zeros((16,), jnp.int32) + i


def _sc_body(xw2, srcp, dstp, ewp, b2, out,
             srcg, dstg, nrmg, disd, rows0, rows1, rows2, disb_v, bias_v,
             deg_sh, acc_sh, gsem, ssem, psem):
    cid = lax.axis_index("c")
    sid = lax.axis_index("s")
    coff = cid * NP
    rows = (rows0, rows1, rows2)

    pltpu.sync_copy(b2.at[cid], bias_v)

    def _fill_ones(i, _):
        disb_v[pl.ds(i * 16, 16)] = jnp.ones((16,), jnp.float32)
        return 0
    lax.fori_loop(0, B // 16, _fill_ones, 0)
    for k in range(SB):
        pltpu.sync_copy(disb_v, deg_sh.at[pl.ds(sid * RPT + k * B, B)])

    def _bias_row(i, _):
        for k in range(H // 16):
            sl = pl.ds(k * 16, 16)
            rows0[i, sl] = bias_v[sl]
        return 0
    lax.fori_loop(0, B, _bias_row, 0)
    for k in range(SB):
        pltpu.sync_copy(rows0, acc_sh.at[pl.ds(sid * RPT + k * B, B)])
    plsc.subcore_barrier()

    def _deg_group(g, _):
        d1 = pltpu.async_copy(dstp.at[sid, g], dstg, psem)
        d2 = pltpu.async_copy(ewp.at[sid, g], nrmg, psem)
        d1.wait()
        d2.wait()
        descs = [pltpu.async_copy(nrmg.at[j], deg_sh.at[dstg.at[j]], psem,
                                  add=True)
                 for j in range(G)]
        for d in descs:
            d.wait()
        return 0
    lax.fori_loop(0, NG, _deg_group, 0)
    plsc.subcore_barrier()

    for k in range(SB):
        off = sid * RPT + k * B
        pltpu.sync_copy(deg_sh.at[pl.ds(off, B)], disb_v)

        def _newton(i, _):
            d = disb_v[pl.ds(i * 16, 16)]
            bits = lax.bitcast_convert_type(d, jnp.int32)
            y = lax.bitcast_convert_type(
                jnp.int32(0x5F3759DF) - (bits >> 1), jnp.float32)
            hd = d * jnp.float32(-0.5)
            for _ in range(3):
                y = y * (hd * y * y + jnp.float32(1.5))
            disb_v[pl.ds(i * 16, 16)] = y
            return 0
        lax.fori_loop(0, B // 16, _newton, 0)
        pltpu.sync_copy(disb_v, deg_sh.at[pl.ds(off, B)])
    plsc.subcore_barrier()

    def _run_pipeline(nb):
        def _scale(j, buf):
            def _scale_row(i, _):
                for u in range(4):
                    r = 4 * i + u
                    nb_ = plsc.load_gather(nrmg,
                                           [_bcast16(j), _bcast16(r)])
                    for k in range(H // 16):
                        sl = pl.ds(k * 16, 16)
                        buf[r, sl] = buf[r, sl] * nb_
                return 0
            lax.fori_loop(0, B // 4, _scale_row, 0)

        gat = {0: pltpu.async_copy(xw2.at[srcg.at[0]], rows[0], gsem)}
        if nb > 1:
            gat[1] = pltpu.async_copy(xw2.at[srcg.at[1]], rows[1], gsem)
        sca = {}
        for j in range(nb):
            cur = rows[j % 3]
            gat[j].wait()
            if j + 2 < nb:
                if j - 1 >= 0:
                    sca[j - 1].wait()
                gat[j + 2] = pltpu.async_copy(
                    xw2.at[srcg.at[j + 2]], rows[(j + 2) % 3], gsem)
            _scale(j, cur)
            sca[j] = pltpu.async_copy(cur, acc_sh.at[dstg.at[j]], ssem,
                                      add=True)
        for j in range(max(0, nb - 3), nb):
            sca[j].wait()

    def _group(g, _):
        d1 = pltpu.async_copy(srcp.at[sid, g], srcg, psem)
        d2 = pltpu.async_copy(dstp.at[sid, g], dstg, psem)
        d3 = pltpu.async_copy(ewp.at[sid, g], nrmg, psem)
        d1.wait()
        d2.wait()
        d3.wait()

        descs = [pltpu.async_copy(deg_sh.at[dstg.at[j]], disd.at[j], psem)
                 for j in range(G)]
        for d in descs:
            d.wait()

        def _mul_dst(j, _):
            for k in range(B // 16):
                sl = pl.ds(k * 16, 16)
                nrmg[j, sl] = nrmg[j, sl] * disd[j, sl]
            return 0
        lax.fori_loop(0, G, _mul_dst, 0)

        descs = [pltpu.async_copy(deg_sh.at[srcg.at[j]], disd.at[j], psem)
                 for j in range(G)]
        for d in descs:
            d.wait()

        def _mul_src(j, _):
            for k in range(B // 16):
                sl = pl.ds(k * 16, 16)
                nrmg[j, sl] = nrmg[j, sl] * disd[j, sl]
                srcg[j, sl] = srcg[j, sl] + coff
            return 0
        lax.fori_loop(0, G, _mul_src, 0)

        _run_pipeline(G)
        return 0
    lax.fori_loop(0, NG, _group, 0)

    descs = [pltpu.async_copy(deg_sh.at[pl.ds(sid * RPT + j * B, B)],
                              disd.at[j], psem)
             for j in range(SB)]
    for d in descs:
        d.wait()

    def _self_stage(j, _):
        base = sid * RPT + j * B
        for k in range(B // 16):
            sl = pl.ds(k * 16, 16)
            idx = lax.iota(jnp.int32, 16) + (base + k * 16)
            dstg[j, sl] = idx
            srcg[j, sl] = idx + coff
            nrmg[j, sl] = disd[j, sl] * disd[j, sl]
        return 0
    lax.fori_loop(0, SB, _self_stage, 0)
    _run_pipeline(SB)
    plsc.subcore_barrier()

    @pl.when(sid < NS - 1)
    def _full_tile():
        pltpu.sync_copy(acc_sh.at[pl.ds(sid * RPT, RPT)],
                        out.at[pl.ds(sid * RPT, RPT), pl.ds(cid * H, H)])

    @pl.when(sid == NS - 1)
    def _last_tile():
        pltpu.sync_copy(acc_sh.at[pl.ds((NS - 1) * RPT, LAST)],
                        out.at[pl.ds((NS - 1) * RPT, LAST),
                               pl.ds(cid * H, H)])


@jax.jit
def kernel(x, edge_index, edge_attr, W, b):
    xw2 = _matmul_halves(x, W).reshape(2 * NP, H)
    srcp = edge_index[0].reshape(NS, NG, G, B)
    dstp = edge_index[1].reshape(NS, NG, G, B)
    ewp = edge_attr.reshape(NS, NG, G, B)
    b2 = b.reshape(2, H)

    mesh = plsc.VectorSubcoreMesh(core_axis_name="c", subcore_axis_name="s",
                                  num_cores=NC, num_subcores=NS)
    sc_fn = pl.kernel(
        _sc_body,
        out_type=jax.ShapeDtypeStruct((N, D_OUT), jnp.float32),
        mesh=mesh,
        compiler_params=pltpu.CompilerParams(needs_layout_passes=False),
        scratch_types=[
            pltpu.VMEM((G, B), jnp.int32),
            pltpu.VMEM((G, B), jnp.int32),
            pltpu.VMEM((G, B), jnp.float32),
            pltpu.VMEM((G, B), jnp.float32),
            pltpu.VMEM((B, H), jnp.float32),
            pltpu.VMEM((B, H), jnp.float32),
            pltpu.VMEM((B, H), jnp.float32),
            pltpu.VMEM((B,), jnp.float32),
            pltpu.VMEM((H,), jnp.float32),
            pltpu.VMEM_SHARED((NP,), jnp.float32),
            pltpu.VMEM_SHARED((NP, H), jnp.float32),
            pltpu.SemaphoreType.DMA,
            pltpu.SemaphoreType.DMA,
            pltpu.SemaphoreType.DMA,
        ],
    )
    return sc_fn(xw2, srcp, dstp, ewp, b2)


# speedup vs baseline: 17.1012x; 1.0258x over previous
import jax
import jax.numpy as jnp
from jax import lax
from jax.experimental import pallas as pl
from jax.experimental.pallas import tpu as pltpu
from jax.experimental.pallas import tpu_sc as plsc

N = 10000
E = 160000
D_IN = 256
D_OUT = 256
H = D_OUT // 2
NC = 2
NS = 16
ET = E // NS
B = 80
G = 25
NG = ET // (G * B)
NP = 10240
RPT = NP // NS
SB = RPT // B
LAST = N - 15 * RPT


def _mm_body(x_ref, w_ref, o_ref):
    o_ref[0] = jnp.dot(x_ref[...], w_ref[...],
                       preferred_element_type=jnp.float32)


def _matmul_halves(x, W):
    bn = 2048
    return pl.pallas_call(
        _mm_body,
        grid=(NP // bn, 2),
        in_specs=[
            pl.BlockSpec((bn, D_IN), lambda i, c: (i, 0)),
            pl.BlockSpec((D_IN, H), lambda i, c: (0, c)),
        ],
        out_specs=pl.BlockSpec((1, bn, H), lambda i, c: (c, i, 0)),
        out_shape=jax.ShapeDtypeStruct((2, NP, H), jnp.float32),
    )(x, W)


def _bcast16(i):
    return jnp.zeros((16,), jnp.int32) + i


def _sc_body(xw2, srcp, dstp, ewp, b2, out,
             srcg, dstg, nrmg, disd, rows0, rows1, rows2, disb_v, bias_v,
             deg_sh, acc_sh, gsem, ssem, psem):
    cid = lax.axis_index("c")
    sid = lax.axis_index("s")
    coff = cid * NP
    rows = (rows0, rows1, rows2)

    pltpu.sync_copy(b2.at[cid], bias_v)

    def _fill_ones(i, _):
        disb_v[pl.ds(i * 16, 16)] = jnp.ones((16,), jnp.float32)
        return 0
    lax.fori_loop(0, B // 16, _fill_ones, 0)
    for k in range(SB):
        pltpu.sync_copy(disb_v, deg_sh.at[pl.ds(sid * RPT + k * B, B)])

    def _bias_row(i, _):
        for k in range(H // 16):
            sl = pl.ds(k * 16, 16)
            rows0[i, sl] = bias_v[sl]
        return 0
    lax.fori_loop(0, B, _bias_row, 0)
    for k in range(SB):
        pltpu.sync_copy(rows0, acc_sh.at[pl.ds(sid * RPT + k * B, B)])
    plsc.subcore_barrier()

    def _deg_group(g, _):
        d1 = pltpu.async_copy(dstp.at[sid, g], dstg, psem)
        d2 = pltpu.async_copy(ewp.at[sid, g], nrmg, psem)
        d1.wait()
        d2.wait()
        descs = [pltpu.async_copy(nrmg.at[j], deg_sh.at[dstg.at[j]], psem,
                                  add=True)
                 for j in range(G)]
        for d in descs:
            d.wait()
        return 0
    lax.fori_loop(0, NG, _deg_group, 0)
    plsc.subcore_barrier()

    for k in range(SB):
        off = sid * RPT + k * B
        pltpu.sync_copy(deg_sh.at[pl.ds(off, B)], disb_v)

        def _newton(i, _):
            d = disb_v[pl.ds(i * 16, 16)]
            bits = lax.bitcast_convert_type(d, jnp.int32)
            y = lax.bitcast_convert_type(
                jnp.int32(0x5F3759DF) - (bits >> 1), jnp.float32)
            hd = d * jnp.float32(-0.5)
            for _ in range(3):
                y = y * (hd * y * y + jnp.float32(1.5))
            disb_v[pl.ds(i * 16, 16)] = y
            return 0
        lax.fori_loop(0, B // 16, _newton, 0)
        pltpu.sync_copy(disb_v, deg_sh.at[pl.ds(off, B)])
    plsc.subcore_barrier()

    def _run_pipeline(nb):
        def _scale(j, buf):
            def _scale_row(i, _):
                for u in range(4):
                    r = 4 * i + u
                    nb_ = plsc.load_gather(nrmg,
                                           [_bcast16(j), _bcast16(r)])
                    for k in range(H // 16):
                        sl = pl.ds(k * 16, 16)
                        buf[r, sl] = buf[r, sl] * nb_
                return 0
            lax.fori_loop(0, B // 4, _scale_row, 0)

        gat = {0: pltpu.async_copy(xw2.at[srcg.at[0]], rows[0], gsem)}
        if nb > 1:
            gat[1] = pltpu.async_copy(xw2.at[srcg.at[1]], rows[1], gsem)
        sca = {}
        for j in range(nb):
            cur = rows[j % 3]
            gat[j].wait()
            if j + 2 < nb:
                if j - 1 >= 0:
                    sca[j - 1].wait()
                gat[j + 2] = pltpu.async_copy(
                    xw2.at[srcg.at[j + 2]], rows[(j + 2) % 3], gsem)
            _scale(j, cur)
            sca[j] = pltpu.async_copy(cur, acc_sh.at[dstg.at[j]], ssem,
                                      add=True)
        for j in range(max(0, nb - 3), nb):
            sca[j].wait()

    def _group(g, _):
        d1 = pltpu.async_copy(srcp.at[sid, g], srcg, psem)
        d2 = pltpu.async_copy(dstp.at[sid, g], dstg, psem)
        d3 = pltpu.async_copy(ewp.at[sid, g], nrmg, psem)
        d1.wait()
        d2.wait()
        d3.wait()

        descs = [pltpu.async_copy(deg_sh.at[dstg.at[j]], disd.at[j], psem)
                 for j in range(G)]
        for d in descs:
            d.wait()

        def _mul_dst(j, _):
            for k in range(B // 16):
                sl = pl.ds(k * 16, 16)
                nrmg[j, sl] = nrmg[j, sl] * disd[j, sl]
            return 0
        lax.fori_loop(0, G, _mul_dst, 0)

        descs = [pltpu.async_copy(deg_sh.at[srcg.at[j]], disd.at[j], psem)
                 for j in range(G)]
        for d in descs:
            d.wait()

        def _mul_src(j, _):
            for k in range(B // 16):
                sl = pl.ds(k * 16, 16)
                nrmg[j, sl] = nrmg[j, sl] * disd[j, sl]
                srcg[j, sl] = srcg[j, sl] + coff
            return 0
        lax.fori_loop(0, G, _mul_src, 0)

        _run_pipeline(G)
        return 0
    lax.fori_loop(0, NG, _group, 0)

    descs = [pltpu.async_copy(deg_sh.at[pl.ds(sid * RPT + j * B, B)],
                              disd.at[j], psem)
             for j in range(SB)]
    for d in descs:
        d.wait()

    def _self_stage(j, _):
        base = sid * RPT + j * B
        for k in range(B // 16):
            sl = pl.ds(k * 16, 16)
            idx = lax.iota(jnp.int32, 16) + (base + k * 16)
            dstg[j, sl] = idx
            srcg[j, sl] = idx + coff
            nrmg[j, sl] = disd[j, sl] * disd[j, sl]
        return 0
    lax.fori_loop(0, SB, _self_stage, 0)
    _run_pipeline(SB)
    plsc.subcore_barrier()

    @pl.when(sid < NS - 1)
    def _full_tile():
        pltpu.sync_copy(acc_sh.at[pl.ds(sid * RPT, RPT)],
                        out.at[pl.ds(sid * RPT, RPT), pl.ds(cid * H, H)])

    @pl.when(sid == NS - 1)
    def _last_tile():
        pltpu.sync_copy(acc_sh.at[pl.ds((NS - 1) * RPT, LAST)],
                        out.at[pl.ds((NS - 1) * RPT, LAST),
                               pl.ds(cid * H, H)])


@jax.jit
def kernel(x, edge_index, edge_attr, W, b):
    xw2 = _matmul_halves(x, W).reshape(2 * NP, H)
    srcp = edge_index[0].reshape(NS, NG, G, B)
    dstp = edge_index[1].reshape(NS, NG, G, B)
    ewp = edge_attr.reshape(NS, NG, G, B)
    b2 = b.reshape(2, H)

    mesh = plsc.VectorSubcoreMesh(core_axis_name="c", subcore_axis_name="s",
                                  num_cores=NC, num_subcores=NS)
    sc_fn = pl.kernel(
        _sc_body,
        out_type=jax.ShapeDtypeStruct((N, D_OUT), jnp.float32),
        mesh=mesh,
        compiler_params=pltpu.CompilerParams(needs_layout_passes=False),
        scratch_types=[
            pltpu.VMEM((G, B), jnp.int32),
            pltpu.VMEM((G, B), jnp.int32),
            pltpu.VMEM((G, B), jnp.float32),
            pltpu.VMEM((G, B), jnp.float32),
            pltpu.VMEM((B, H), jnp.float32),
            pltpu.VMEM((B, H), jnp.float32),
            pltpu.VMEM((B, H), jnp.float32),
            pltpu.VMEM((B,), jnp.float32),
            pltpu.VMEM((H,), jnp.float32),
            pltpu.VMEM_SHARED((NP,), jnp.float32),
            pltpu.VMEM_SHARED((NP, H), jnp.float32),
            pltpu.SemaphoreType.DMA,
            pltpu.SemaphoreType.DMA,
            pltpu.SemaphoreType.DMA,
        ],
    )
    return sc_fn(xw2, srcp, dstp, ewp, b2)
